# Initial kernel scaffold; baseline (speedup 1.0000x reference)
import jax
import jax.numpy as jnp
from jax.experimental import pallas as pl


def kernel(input_ids, token_table, pos_table):
    raise NotImplementedError("write your pallas kernel here")


# speedup vs baseline: 2.0465x; 2.0465x over previous
import functools

import jax
import jax.numpy as jnp
from jax import lax
from jax.experimental import pallas as pl
from jax.experimental.pallas import tpu as pltpu
from jax.experimental.pallas import tpu_sc as plsc

VOCAB = 100000
D = 64
B, S = 1024, 200
NW = 32
TOK_PER_W = B * S // NW
SEQ_PER_W = TOK_PER_W // S
HALF = 100

_mesh = plsc.VectorSubcoreMesh(core_axis_name="c", subcore_axis_name="s")


@functools.partial(
    pl.kernel,
    mesh=_mesh,
    out_type=jax.ShapeDtypeStruct((B * S, D), jnp.float32),
    scratch_types=[
        pltpu.VMEM((SEQ_PER_W * 2, HALF), jnp.int32),
        pltpu.VMEM((TOK_PER_W + 16,), jnp.int32),
        pltpu.VMEM((S, D), jnp.float32),
        pltpu.VMEM((S, D), jnp.float32),
        pltpu.SemaphoreType.DMA,
    ],
    compiler_params=pltpu.CompilerParams(use_tc_tiling_on_sc=False),
)
def _encoder(ids_stream, ids_chk, table, pos, out, idx_v, chk_v, pos_v, buf_v, sem):
    w = lax.axis_index("s") * 2 + lax.axis_index("c")
    pltpu.sync_copy(ids_stream.at[w], idx_v)
    pltpu.sync_copy(ids_chk.at[w], chk_v.at[pl.ds(0, TOK_PER_W)])
    pltpu.sync_copy(pos.at[pl.ds(0, S)], pos_v)

    def _mn(i, acc):
        return jnp.minimum(acc, chk_v[pl.ds(i * 16, 16)])

    acc = lax.fori_loop(0, TOK_PER_W // 16, _mn,
                        jnp.full((16,), jnp.iinfo(jnp.int32).max, jnp.int32))
    lanes = lax.iota(jnp.int32, 16)
    for shift in (8, 4, 2, 1):
        perm = lax.rem(lanes + shift, 16)
        g = lax.gather(
            acc, perm[:, None],
            dimension_numbers=lax.GatherDimensionNumbers(
                offset_dims=(), collapsed_slice_dims=(0,), start_index_map=(0,)),
            slice_sizes=(1,), mode=lax.GatherScatterMode.PROMISE_IN_BOUNDS)
        acc = jnp.minimum(acc, g)
    has_pad = acc[0] == 0

    def _chunk(cidx, carry):
        base = cidx * S
        cp1 = pltpu.async_copy(table.at[idx_v.at[2 * cidx]],
                               buf_v.at[pl.ds(0, HALF)], sem)
        cp2 = pltpu.async_copy(table.at[idx_v.at[2 * cidx + 1]],
                               buf_v.at[pl.ds(HALF, HALF)], sem)
        cp1.wait()
        cp2.wait()

        @pl.when(has_pad)
        def _():
            def _fix(r, c2):
                idv = chk_v[pl.ds(base + r, 16)]
                @pl.when(idv[0] == 0)
                def _():
                    zero = jnp.zeros((16,), jnp.float32)
                    for k in range(4):
                        buf_v[r, pl.ds(k * 16, 16)] = zero
                return c2
            lax.fori_loop(0, S, _fix, 0)

        def _add(r, c2):
            for k in range(4):
                sl = pl.ds(k * 16, 16)
                buf_v[r, sl] = buf_v[r, sl] + pos_v[r, sl]
            return c2
        lax.fori_loop(0, S, _add, 0)

        pltpu.sync_copy(buf_v, out.at[pl.ds(w * TOK_PER_W + base, S)])
        return carry

    lax.fori_loop(0, SEQ_PER_W, _chunk, 0)


def kernel(input_ids, token_table, pos_table):
    ids = input_ids.astype(jnp.int32)
    ids_stream = ids.reshape(NW, SEQ_PER_W * 2, HALF)
    ids_chk = ids.reshape(NW, TOK_PER_W)
    out = _encoder(ids_stream, ids_chk, token_table, pos_table)
    return out.reshape(B, S, D)
